# baseline (speedup 1.0000x reference)
import jax
import jax.numpy as jnp
from jax import lax
from jax.experimental import pallas as pl
from jax.experimental.pallas import tpu as pltpu

_TBL_H = 15
_TBL_W = 15
_BLOCK = 256


def _gather_bias_kernel(kflat_ref, tbl_ref, bias_ref):
    kflat = kflat_ref[...]

    def body(t, acc):
        v = tbl_ref[t // _TBL_W, t % _TBL_W]
        return acc + jnp.where(kflat == t, v, 0.0)

    bias_ref[...] = lax.fori_loop(
        0, _TBL_H * _TBL_W, body, jnp.zeros(kflat.shape, jnp.float32)
    )


def _add_kernel(bias_ref, x_ref, o_ref):
    o_ref[...] = x_ref[...] + bias_ref[...]


def kernel(x, pos_embeddings, relative_indices):
    nb, nh, h, w = x.shape
    n = nb * nh
    hw = h * w
    x2 = x.reshape(n, hw)
    kflat = (relative_indices[:, :, 0] * _TBL_W + relative_indices[:, :, 1]).reshape(1, hw)

    bias = pl.pallas_call(
        _gather_bias_kernel,
        in_specs=[
            pl.BlockSpec((1, hw), lambda: (0, 0)),
            pl.BlockSpec(memory_space=pltpu.SMEM),
        ],
        out_specs=pl.BlockSpec((1, hw), lambda: (0, 0)),
        out_shape=jax.ShapeDtypeStruct((1, hw), jnp.float32),
    )(kflat, pos_embeddings)

    out = pl.pallas_call(
        _add_kernel,
        grid=(n // _BLOCK,),
        in_specs=[
            pl.BlockSpec((1, hw), lambda i: (0, 0)),
            pl.BlockSpec((_BLOCK, hw), lambda i: (i, 0)),
        ],
        out_specs=pl.BlockSpec((_BLOCK, hw), lambda i: (i, 0)),
        out_shape=jax.ShapeDtypeStruct((n, hw), jnp.float32),
        compiler_params=pltpu.CompilerParams(
            dimension_semantics=(pltpu.ARBITRARY,),
        ),
    )(bias, x2)
    return out.reshape(x.shape)
